# Initial kernel scaffold; baseline (speedup 1.0000x reference)
import jax
import jax.numpy as jnp
from jax.experimental import pallas as pl


def kernel(x, table, pe):
    raise NotImplementedError("write your pallas kernel here")


# speedup vs baseline: 2.3456x; 2.3456x over previous
import functools

import jax
import jax.numpy as jnp
from jax import lax
from jax.experimental import pallas as pl
from jax.experimental.pallas import tpu as pltpu
from jax.experimental.pallas import tpu_sc as plsc

D = 64
CHUNK = 128
LANES = 16


@functools.partial(jax.jit, static_argnames=("n_workers", "n_chunks", "seq"))
def _sc_embed_add(idx, table, pe_ext, *, n_workers, n_chunks, seq):
    n_rows = n_workers * n_chunks * CHUNK
    rows_w = n_chunks * CHUNK
    mesh = plsc.VectorSubcoreMesh(core_axis_name="c", subcore_axis_name="s")
    num_cores = mesh.num_cores

    @functools.partial(
        pl.kernel,
        out_type=jax.ShapeDtypeStruct((n_rows, D), jnp.float32),
        mesh=mesh,
        scratch_types=[
            pltpu.VMEM((n_chunks, CHUNK), jnp.int32),
            pltpu.VMEM((seq + CHUNK, D), jnp.float32),
            pltpu.VMEM((CHUNK, D), jnp.float32),
            pltpu.SemaphoreType.DMA,
        ],
        compiler_params=pltpu.CompilerParams(use_tc_tiling_on_sc=False),
    )
    def k(idx_hbm, table_hbm, pe_hbm, out_hbm, idx_v, pe_v, rows_v, sem):
        wid = lax.axis_index("s") * num_cores + lax.axis_index("c")
        pltpu.sync_copy(idx_hbm.at[wid], idx_v)
        pltpu.sync_copy(pe_hbm, pe_v)
        base = wid * rows_w

        @pl.loop(0, n_chunks)
        def chunk_body(c):
            pltpu.async_copy(table_hbm.at[idx_v.at[c]], rows_v, sem).wait()
            s0 = lax.rem(c * CHUNK, seq)

            @pl.loop(0, CHUNK)
            def row_body(r):
                for kk in range(D // LANES):
                    v = pe_v[s0 + r, pl.ds(kk * LANES, LANES)]
                    plsc.addupdate(rows_v.at[r, pl.ds(kk * LANES, LANES)], v)

            pltpu.sync_copy(rows_v, out_hbm.at[pl.ds(base + c * CHUNK, CHUNK)])

    return k(idx, table, pe_ext)


def kernel(x, table, pe):
    b, s = x.shape
    n = b * s
    n_workers = 32
    rows_w = n // n_workers
    n_chunks = rows_w // CHUNK
    idx = x.reshape(n_workers, n_chunks, CHUNK)
    pe_rows = pe[0, :s, :]
    pe_ext = jnp.concatenate([pe_rows, pe_rows[:CHUNK]], axis=0)
    out = _sc_embed_add(
        idx, table, pe_ext, n_workers=n_workers, n_chunks=n_chunks, seq=s
    )
    return out.reshape(b, s, D)


# speedup vs baseline: 3.1088x; 1.3254x over previous
import functools

import jax
import jax.numpy as jnp
from jax import lax
from jax.experimental import pallas as pl
from jax.experimental.pallas import tpu as pltpu
from jax.experimental.pallas import tpu_sc as plsc

D = 64
CHUNK = 128
LANES = 16
NBUF = 4


@functools.partial(jax.jit, static_argnames=("n_workers", "n_chunks", "seq"))
def _sc_embed_add(idx, table, pe_ext, *, n_workers, n_chunks, seq):
    n_rows = n_workers * n_chunks * CHUNK
    rows_w = n_chunks * CHUNK
    assert n_chunks % NBUF == 0 and n_chunks >= 2 * NBUF
    mesh = plsc.VectorSubcoreMesh(core_axis_name="c", subcore_axis_name="s")
    num_cores = mesh.num_cores

    @functools.partial(
        pl.kernel,
        out_type=jax.ShapeDtypeStruct((n_rows, D), jnp.float32),
        mesh=mesh,
        scratch_types=[
            pltpu.VMEM((n_chunks, CHUNK), jnp.int32),
            pltpu.VMEM((seq + CHUNK, D), jnp.float32),
        ]
        + [pltpu.VMEM((CHUNK, D), jnp.float32)] * NBUF
        + [pltpu.SemaphoreType.DMA] * (2 * NBUF),
        compiler_params=pltpu.CompilerParams(use_tc_tiling_on_sc=False),
    )
    def k(idx_hbm, table_hbm, pe_hbm, out_hbm, idx_v, pe_v, *bufs):
        rows = bufs[:NBUF]
        gsem = bufs[NBUF:2 * NBUF]
        osem = bufs[2 * NBUF:]
        wid = lax.axis_index("s") * num_cores + lax.axis_index("c")
        pltpu.sync_copy(idx_hbm.at[wid], idx_v)
        pltpu.sync_copy(pe_hbm, pe_v)
        base = wid * rows_w

        def start_gather(b, c):
            pltpu.async_copy(table_hbm.at[idx_v.at[c]], rows[b], gsem[b])

        def wait_gather(b, c):
            pltpu.make_async_copy(
                table_hbm.at[idx_v.at[c]], rows[b], gsem[b]
            ).wait()

        def start_out(b, c):
            pltpu.async_copy(
                rows[b], out_hbm.at[pl.ds(base + c * CHUNK, CHUNK)], osem[b]
            )

        def wait_out(b, c):
            pltpu.make_async_copy(
                rows[b], out_hbm.at[pl.ds(base + c * CHUNK, CHUNK)], osem[b]
            ).wait()

        def add_pe(b, c):
            s0 = lax.rem(c * CHUNK, seq)
            rv = rows[b]

            @pl.loop(0, CHUNK, unroll=8)
            def row_body(r):
                for kk in range(D // LANES):
                    v = pe_v[s0 + r, pl.ds(kk * LANES, LANES)]
                    plsc.addupdate(rv.at[r, pl.ds(kk * LANES, LANES)], v)

        for b in range(NBUF):
            start_gather(b, b)

        for b in range(NBUF):
            wait_gather(b, b)
            add_pe(b, b)
            start_out(b, b)
            start_gather(b, b + NBUF)

        @pl.loop(0, n_chunks // NBUF - 2)
        def outer(o):
            for b in range(NBUF):
                c = NBUF + o * NBUF + b
                wait_gather(b, c)
                add_pe(b, c)
                wait_out(b, c - NBUF)
                start_out(b, c)
                start_gather(b, c + NBUF)

        for b in range(NBUF):
            c = n_chunks - NBUF + b
            wait_gather(b, c)
            add_pe(b, c)
            wait_out(b, c - NBUF)
            start_out(b, c)

        for b in range(NBUF):
            wait_out(b, n_chunks - NBUF + b)

    return k(idx, table, pe_ext)


def kernel(x, table, pe):
    b, s = x.shape
    n = b * s
    n_workers = 32
    rows_w = n // n_workers
    n_chunks = rows_w // CHUNK
    idx = x.reshape(n_workers, n_chunks, CHUNK)
    pe_rows = pe[0, :s, :]
    pe_ext = jnp.concatenate([pe_rows, pe_rows[:CHUNK]], axis=0)
    out = _sc_embed_add(
        idx, table, pe_ext, n_workers=n_workers, n_chunks=n_chunks, seq=s
    )
    return out.reshape(b, s, D)
